# baseline (speedup 1.0000x reference)
import jax
import jax.numpy as jnp
from jax.experimental import pallas as pl

_NUM_CLASSES = 2
_NUM_ANCHORS = 10
_HB = 64


def _head_kernel(x_ref, wc_ref, bc_ref, wb_ref, bb_ref, cls_ref, box_ref):
    c = x_ref.shape[2]
    n = x_ref.shape[3] * x_ref.shape[4]
    x = x_ref[0, 0].reshape(c, n)
    dn = (((1,), (0,)), ((), ()))
    cls_ref[0] = jax.lax.dot_general(
        wc_ref[0], x, dn, preferred_element_type=jnp.float32) + bc_ref[0]
    box_ref[0] = jax.lax.dot_general(
        wb_ref[0], x, dn, preferred_element_type=jnp.float32) + bb_ref[0]


def kernel(features, Wc, bc, Wb, bb):
    L, B, C, H, W = features.shape
    Oc = Wc.shape[1]
    Ob = Wb.shape[1]
    A = _NUM_ANCHORS
    K = _NUM_CLASSES
    HW = H * W
    N = L * HW * A
    nblk = H // _HB
    NB = _HB * W

    Wcp = Wc.reshape(L, A, K, C).transpose(0, 2, 1, 3).reshape(L, Oc, C)
    bcp = bc.reshape(L, A, K).transpose(0, 2, 1).reshape(L, Oc, 1)
    Wbp = Wb.reshape(L, A, 4, C).transpose(0, 2, 1, 3).reshape(L, Ob, C)
    bbp = bb.reshape(L, A, 4).transpose(0, 2, 1).reshape(L, Ob, 1)

    grid = (L, B, nblk)
    cls_t, box_t = pl.pallas_call(
        _head_kernel,
        grid=grid,
        in_specs=[
            pl.BlockSpec((1, 1, C, _HB, W), lambda l, b, h: (l, b, 0, h, 0)),
            pl.BlockSpec((1, Oc, C), lambda l, b, h: (l, 0, 0)),
            pl.BlockSpec((1, Oc, 1), lambda l, b, h: (l, 0, 0)),
            pl.BlockSpec((1, Ob, C), lambda l, b, h: (l, 0, 0)),
            pl.BlockSpec((1, Ob, 1), lambda l, b, h: (l, 0, 0)),
        ],
        out_specs=[
            pl.BlockSpec((1, Oc, NB), lambda l, b, h: (b, 0, l * nblk + h)),
            pl.BlockSpec((1, Ob, NB), lambda l, b, h: (b, 0, l * nblk + h)),
        ],
        out_shape=[
            jax.ShapeDtypeStruct((B, Oc, L * HW), jnp.float32),
            jax.ShapeDtypeStruct((B, Ob, L * HW), jnp.float32),
        ],
    )(features, Wcp, bcp, Wbp, bbp)

    cls_score = (cls_t.reshape(B, K, A, L * HW)
                 .transpose(0, 3, 2, 1)
                 .reshape(B, N, K))
    bbox_pred = (box_t.reshape(B, 4, A, L * HW)
                 .transpose(0, 3, 2, 1)
                 .reshape(B, N, 4))
    return (cls_score, bbox_pred)


# speedup vs baseline: 1.0383x; 1.0383x over previous
import jax
import jax.numpy as jnp
from jax.experimental import pallas as pl

_NUM_CLASSES = 2
_NUM_ANCHORS = 10
_CB = 128


def _head_kernel(x_ref, wc_ref, bc_ref, wb_ref, bb_ref, cls_ref, box_ref):
    cb = x_ref.shape[2]
    n = x_ref.shape[3] * x_ref.shape[4]
    x = x_ref[0, 0].reshape(cb, n)
    dn = (((1,), (0,)), ((), ()))
    yc = jax.lax.dot_general(wc_ref[0], x, dn,
                             preferred_element_type=jnp.float32)
    yb = jax.lax.dot_general(wb_ref[0], x, dn,
                             preferred_element_type=jnp.float32)
    ci = pl.program_id(2)

    @pl.when(ci == 0)
    def _init():
        cls_ref[0] = yc + bc_ref[0]
        box_ref[0] = yb + bb_ref[0]

    @pl.when(ci != 0)
    def _acc():
        cls_ref[0] += yc
        box_ref[0] += yb


def kernel(features, Wc, bc, Wb, bb):
    L, B, C, H, W = features.shape
    Oc = Wc.shape[1]
    Ob = Wb.shape[1]
    A = _NUM_ANCHORS
    K = _NUM_CLASSES
    HW = H * W
    N = L * HW * A
    ncb = C // _CB

    Wcp = Wc.reshape(L, A, K, C).transpose(0, 2, 1, 3).reshape(L, Oc, C)
    bcp = bc.reshape(L, A, K).transpose(0, 2, 1).reshape(L, Oc, 1)
    Wbp = Wb.reshape(L, A, 4, C).transpose(0, 2, 1, 3).reshape(L, Ob, C)
    bbp = bb.reshape(L, A, 4).transpose(0, 2, 1).reshape(L, Ob, 1)

    grid = (L, B, ncb)
    cls_t, box_t = pl.pallas_call(
        _head_kernel,
        grid=grid,
        in_specs=[
            pl.BlockSpec((1, 1, _CB, H, W), lambda l, b, c: (l, b, c, 0, 0)),
            pl.BlockSpec((1, Oc, _CB), lambda l, b, c: (l, 0, c)),
            pl.BlockSpec((1, Oc, 1), lambda l, b, c: (l, 0, 0)),
            pl.BlockSpec((1, Ob, _CB), lambda l, b, c: (l, 0, c)),
            pl.BlockSpec((1, Ob, 1), lambda l, b, c: (l, 0, 0)),
        ],
        out_specs=[
            pl.BlockSpec((1, Oc, HW), lambda l, b, c: (b, 0, l)),
            pl.BlockSpec((1, Ob, HW), lambda l, b, c: (b, 0, l)),
        ],
        out_shape=[
            jax.ShapeDtypeStruct((B, Oc, L * HW), jnp.float32),
            jax.ShapeDtypeStruct((B, Ob, L * HW), jnp.float32),
        ],
    )(features, Wcp, bcp, Wbp, bbp)

    cls_score = (cls_t.reshape(B, K, A, L * HW)
                 .transpose(0, 3, 2, 1)
                 .reshape(B, N, K))
    bbox_pred = (box_t.reshape(B, 4, A, L * HW)
                 .transpose(0, 3, 2, 1)
                 .reshape(B, N, 4))
    return (cls_score, bbox_pred)


# speedup vs baseline: 1.0388x; 1.0005x over previous
import jax
import jax.numpy as jnp
from jax.experimental import pallas as pl

_NUM_CLASSES = 2
_NUM_ANCHORS = 10
_NCB = 4
_J = 2


def _head_kernel(*refs):
    x_refs = refs[:_J]
    wc_refs = refs[_J:2 * _J]
    wb_refs = refs[2 * _J:3 * _J]
    bc_ref, bb_ref, cls_ref, box_ref = refs[3 * _J:]
    dn = (((1,), (0,)), ((), ()))
    yc = None
    yb = None
    for j in range(_J):
        cb = x_refs[j].shape[2]
        n = x_refs[j].shape[3] * x_refs[j].shape[4]
        x = x_refs[j][0, 0].reshape(cb, n)
        pc = jax.lax.dot_general(wc_refs[j][0, 0], x, dn,
                                 preferred_element_type=jnp.float32)
        pb = jax.lax.dot_general(wb_refs[j][0, 0], x, dn,
                                 preferred_element_type=jnp.float32)
        yc = pc if yc is None else yc + pc
        yb = pb if yb is None else yb + pb
    ci = pl.program_id(2)

    @pl.when(ci == 0)
    def _init():
        cls_ref[0] = yc + bc_ref[0]
        box_ref[0] = yb + bb_ref[0]

    @pl.when(ci != 0)
    def _acc():
        cls_ref[0] += yc
        box_ref[0] += yb


def kernel(features, Wc, bc, Wb, bb):
    L, B, C, H, W = features.shape
    Oc = Wc.shape[1]
    Ob = Wb.shape[1]
    A = _NUM_ANCHORS
    K = _NUM_CLASSES
    HW = H * W
    N = L * HW * A
    cbj = C // (_NCB * _J)

    nsplit = _NCB * _J
    Wcp = (Wc.reshape(L, A, K, C).transpose(0, 2, 1, 3)
           .reshape(L, Oc, nsplit, cbj).transpose(0, 2, 1, 3))
    bcp = bc.reshape(L, A, K).transpose(0, 2, 1).reshape(L, Oc, 1)
    Wbp = (Wb.reshape(L, A, 4, C).transpose(0, 2, 1, 3)
           .reshape(L, Ob, nsplit, cbj).transpose(0, 2, 1, 3))
    bbp = bb.reshape(L, A, 4).transpose(0, 2, 1).reshape(L, Ob, 1)

    x_specs = [
        pl.BlockSpec((1, 1, cbj, H, W),
                     lambda l, b, c, j=j: (l, b, c * _J + j, 0, 0))
        for j in range(_J)
    ]
    wc_specs = [
        pl.BlockSpec((1, 1, Oc, cbj),
                     lambda l, b, c, j=j: (l, c * _J + j, 0, 0))
        for j in range(_J)
    ]
    wb_specs = [
        pl.BlockSpec((1, 1, Ob, cbj),
                     lambda l, b, c, j=j: (l, c * _J + j, 0, 0))
        for j in range(_J)
    ]

    grid = (L, B, _NCB)
    cls_t, box_t = pl.pallas_call(
        _head_kernel,
        grid=grid,
        in_specs=x_specs + wc_specs + wb_specs + [
            pl.BlockSpec((1, Oc, 1), lambda l, b, c: (l, 0, 0)),
            pl.BlockSpec((1, Ob, 1), lambda l, b, c: (l, 0, 0)),
        ],
        out_specs=[
            pl.BlockSpec((1, Oc, HW), lambda l, b, c: (b, 0, l)),
            pl.BlockSpec((1, Ob, HW), lambda l, b, c: (b, 0, l)),
        ],
        out_shape=[
            jax.ShapeDtypeStruct((B, Oc, L * HW), jnp.float32),
            jax.ShapeDtypeStruct((B, Ob, L * HW), jnp.float32),
        ],
    )(*([features] * _J), *([Wcp] * _J), *([Wbp] * _J), bcp, bbp)

    cls_score = (cls_t.reshape(B, K, A, L * HW)
                 .transpose(0, 3, 2, 1)
                 .reshape(B, N, K))
    bbox_pred = (box_t.reshape(B, 4, A, L * HW)
                 .transpose(0, 3, 2, 1)
                 .reshape(B, N, 4))
    return (cls_score, bbox_pred)
